# baseline (speedup 1.0000x reference)
import jax
import jax.numpy as jnp
from jax import lax
from jax.experimental import pallas as pl
from jax.experimental.pallas import tpu as pltpu
from jax.experimental.pallas import tpu_sc as plsc
from jax._src.pallas import mpmd as _mpmd

NUM_NODES = 100000
B = 4096
B2 = 2 * B
B3 = 3 * B
DE = 128
DMAIL = 144
DT = 100
NW = 32
CHUNK = 128
RN = 1024
NSTEPS = 24
WBLK = 512
JBLK = 2048

_f32 = jnp.float32
_i32 = jnp.int32


def _k1_body(mem_hbm, mail_hbm, mem_ts_hbm, mail_ts_hbm, idx_hbm,
             gmem, gmail, gmem_ts, gmail_ts,
             idx_v, idxp_v, rows_a, mrow_v, ts_v, ts2_v, sem, rsem):
    wid = lax.axis_index("s") * 2 + lax.axis_index("c")
    per_tile = B3 // NW
    base = wid * per_tile
    for j in range(per_tile // CHUNK):
        off = base + j * CHUNK
        pltpu.sync_copy(idx_hbm.at[pl.ds(off, CHUNK)], idx_v)
        pltpu.sync_copy(idx_hbm.at[pl.ds(off, CHUNK)],
                        idxp_v.at[pl.ds(0, CHUNK)])
        a = pltpu.async_copy(mem_hbm.at[idx_v], rows_a, sem)
        c = pltpu.async_copy(mem_ts_hbm.at[idx_v], ts_v, sem)
        d = pltpu.async_copy(mail_ts_hbm.at[idx_v], ts2_v, sem)

        def fire(r, _):
            idx = idxp_v[pl.ds(r, 16)][0]
            pltpu.make_async_copy(
                mail_hbm.at[pl.ds(idx, 1), pl.ds(0, DE)],
                mrow_v.at[pl.ds(r, 1), pl.ds(0, DE)],
                rsem,
            ).start()
            pltpu.make_async_copy(
                mail_hbm.at[pl.ds(idx, 1), pl.ds(DE, DMAIL - DE)],
                mrow_v.at[pl.ds(r, 1), pl.ds(DE, DMAIL - DE)],
                rsem,
            ).start()
            return 0

        lax.fori_loop(0, CHUNK, fire, 0)

        def drain(r, _):
            idx = idxp_v[pl.ds(r, 16)][0]
            pltpu.make_async_copy(
                mail_hbm.at[pl.ds(idx, 1), pl.ds(0, DE)],
                mrow_v.at[pl.ds(r, 1), pl.ds(0, DE)],
                rsem,
            ).wait()
            pltpu.make_async_copy(
                mail_hbm.at[pl.ds(idx, 1), pl.ds(DE, DMAIL - DE)],
                mrow_v.at[pl.ds(r, 1), pl.ds(DE, DMAIL - DE)],
                rsem,
            ).wait()
            return 0

        lax.fori_loop(0, CHUNK, drain, 0)
        a.wait()
        c.wait()
        d.wait()
        pltpu.sync_copy(rows_a, gmem.at[pl.ds(off, CHUNK)])
        pltpu.sync_copy(mrow_v, gmail.at[pl.ds(off, CHUNK), :])
        pltpu.sync_copy(ts_v, gmem_ts.at[pl.ds(off, CHUNK)])
        pltpu.sync_copy(ts2_v, gmail_ts.at[pl.ds(off, CHUNK)])


_k1 = pl.kernel(
    _k1_body,
    out_type=(
        jax.ShapeDtypeStruct((B3, DE), _f32),
        jax.ShapeDtypeStruct((B3, DMAIL), _f32),
        jax.ShapeDtypeStruct((B3,), _f32),
        jax.ShapeDtypeStruct((B3,), _f32),
    ),
    mesh=plsc.VectorSubcoreMesh(core_axis_name="c", subcore_axis_name="s"),
    scratch_types=(
        pltpu.VMEM((CHUNK,), _i32),
        pltpu.VMEM((CHUNK + 16,), _i32),
        pltpu.VMEM((CHUNK, DE), _f32),
        pltpu.VMEM((CHUNK, DMAIL), _f32),
        pltpu.VMEM((CHUNK,), _f32),
        pltpu.VMEM((CHUNK,), _f32),
        pltpu.SemaphoreType.DMA,
        pltpu.SemaphoreType.DMA,
    ),
    name="atlas_k1_gather",
)


def _k02_body(mem_in, mail_in, mem_ts_in, mail_ts_in,
              gmem, gmail, gmem_ts, gmail_ts, root_ts, edge_feat,
              dstB_ref, dstC_ref,
              w_ih, bsum, w_hh, time_w, time_b, tl_w, tl_b, ln_g, ln_b,
              ep_srcT, ep_src_b, ep_dstT, ep_dst_b, ep_out_w, ep_out_b,
              out_mem, out_mail, out_mem_ts, out_mail_ts,
              norm8k, mailrow, loser_out, pos_out, neg_out,
              norm_scr, proj_scr, sem1, sem2, sem3, sem4):
    g = pl.program_id(0)

    @pl.when(g == 0)
    def _fire_copies():
        pltpu.make_async_copy(mem_in, out_mem, sem1).start()
        pltpu.make_async_copy(mail_in, out_mail, sem2).start()
        pltpu.make_async_copy(mem_ts_in, out_mem_ts, sem3).start()
        pltpu.make_async_copy(mail_ts_in, out_mail_ts, sem4).start()

    @pl.when(g == NSTEPS - 1)
    def _wait_copies():
        pltpu.make_async_copy(mem_in, out_mem, sem1).wait()
        pltpu.make_async_copy(mail_in, out_mail, sem2).wait()
        pltpu.make_async_copy(mem_ts_in, out_mem_ts, sem3).wait()
        pltpu.make_async_copy(mail_ts_in, out_mail_ts, sem4).wait()

    @pl.when(g < B3 // RN)
    def _dense():
        prev_mem = gmem[...]
        m = gmail[...]
        prev_ts = gmem_ts[...]
        m_ts = gmail_ts[...]
        dt = m_ts - prev_ts
        tf = jnp.cos(dt[:, None] * time_w[...] + time_b[...])
        x = jnp.concatenate([m, tf], axis=1)
        up = jnp.tanh(
            jnp.dot(x, w_ih[...], preferred_element_type=_f32)
            + jnp.dot(prev_mem, w_hh[...], preferred_element_type=_f32)
            + bsum[...]
        )
        mu = jnp.mean(up, axis=1, keepdims=True)
        var = jnp.mean((up - mu) ** 2, axis=1, keepdims=True)
        norm = (up - mu) / jnp.sqrt(var + 1e-5) * ln_g[...] + ln_b[...]
        times = root_ts[...]
        tidiff = (times - m_ts) / (times + 1.0)
        proj = norm * (1.0 + tidiff[:, None] * tl_w[...] + tl_b[...])
        norm_scr[pl.ds(g * RN, RN), :] = norm
        proj_scr[pl.ds(g * RN, RN), :] = proj

        @pl.when(g < B2 // RN)
        def _():
            norm8k[...] = norm

    @pl.when((g >= 12) & (g < 20))
    def _mailrow():
        k = g - 12
        rows0 = k * RN
        src_off = jnp.where(rows0 < B, rows0 + B, rows0 - B)
        me = norm_scr[pl.ds(src_off, RN), :]
        mailrow[...] = jnp.concatenate([me, edge_feat[...]], axis=1)

    @pl.when((g >= 20) & (g < 24))
    def _scores():
        q = g - 20
        r0 = q * RN
        src = proj_scr[pl.ds(r0, RN), :]
        dst = proj_scr[pl.ds(B + r0, RN), :]
        neg = proj_scr[pl.ds(2 * B + r0, RN), :]
        sx = jnp.dot(src, ep_srcT[...], preferred_element_type=_f32) + ep_src_b[...]
        hp = jnp.maximum(sx + jnp.dot(dst, ep_dstT[...], preferred_element_type=_f32) + ep_dst_b[...], 0.0)
        hn = jnp.maximum(sx + jnp.dot(neg, ep_dstT[...], preferred_element_type=_f32) + ep_dst_b[...], 0.0)
        pos_out[pl.ds(r0, RN), :] = jnp.dot(hp, ep_out_w[...], preferred_element_type=_f32) + ep_out_b[...]
        neg_out[pl.ds(r0, RN), :] = jnp.dot(hn, ep_out_w[...], preferred_element_type=_f32) + ep_out_b[...]

    @pl.when(g < B2 // WBLK)
    def _loser():
        i0 = g * WBLK
        pos_i = dstB_ref[pl.ds(i0, WBLK), :]
        cd = i0 // JBLK

        pos_jd = dstC_ref[pl.ds(cd, 1), 0, :]
        jd = cd * JBLK + lax.broadcasted_iota(_i32, (WBLK, JBLK), 1)
        icol = i0 + lax.broadcasted_iota(_i32, (WBLK, JBLK), 0)
        acc0 = jnp.any(
            (pos_i == pos_jd) & (jd > icol), axis=1, keepdims=True
        ).astype(_i32)

        def body(c, acc):
            pos_j = dstC_ref[pl.ds(c, 1), 0, :]
            hit = jnp.any(pos_i == pos_j, axis=1, keepdims=True).astype(_i32)
            return jnp.maximum(acc, hit)

        acc = lax.fori_loop(cd + 1, B2 // JBLK, body, acc0)
        loser_out[...] = acc


def _k02(mem, mail, mem_ts, mail_ts, gmem, gmail, gmem_ts, gmail_ts,
         root_ts, edge_feat, dstB, dstC, weights):
    (w_ih, bsum, w_hh, time_w, time_b, tl_w, tl_b, ln_g, ln_b,
     ep_srcT, ep_src_b, ep_dstT, ep_dst_b, ep_out_w, ep_out_b) = weights
    full = lambda shape: pl.BlockSpec(shape, lambda g: tuple(0 for _ in shape))
    dense_i = lambda g: (jnp.minimum(g, B3 // RN - 1), 0)
    dense_i1 = lambda g: (jnp.minimum(g, B3 // RN - 1),)
    return pl.pallas_call(
        _k02_body,
        grid=(NSTEPS,),
        in_specs=[
            pl.BlockSpec(memory_space=pltpu.HBM),
            pl.BlockSpec(memory_space=pltpu.HBM),
            pl.BlockSpec(memory_space=pltpu.HBM),
            pl.BlockSpec(memory_space=pltpu.HBM),
            pl.BlockSpec((RN, DE), dense_i),
            pl.BlockSpec((RN, DMAIL), dense_i),
            pl.BlockSpec((RN,), dense_i1),
            pl.BlockSpec((RN,), dense_i1),
            pl.BlockSpec((RN,), lambda g: (g % 4,)),
            pl.BlockSpec((RN, 16), lambda g: (jnp.clip(g - 12, 0, 7) % 4, 0)),
            pl.BlockSpec((B2, 1), lambda g: (0, 0)),
            pl.BlockSpec((B2 // JBLK, 1, JBLK), lambda g: (0, 0, 0)),
            full((244, DE)),
            full((1, DE)),
            full((DE, DE)),
            full((1, DT)),
            full((1, DT)),
            full((1, DE)),
            full((1, DE)),
            full((1, DE)),
            full((1, DE)),
            full((DE, DE)),
            full((1, DE)),
            full((DE, DE)),
            full((1, DE)),
            full((DE, 1)),
            full((1, 1)),
        ],
        out_specs=[
            pl.BlockSpec(memory_space=pltpu.HBM),
            pl.BlockSpec(memory_space=pltpu.HBM),
            pl.BlockSpec(memory_space=pltpu.HBM),
            pl.BlockSpec(memory_space=pltpu.HBM),
            pl.BlockSpec((RN, DE), lambda g: (jnp.minimum(g, B2 // RN - 1), 0)),
            pl.BlockSpec((RN, DMAIL), lambda g: (jnp.clip(g - 12, 0, 7), 0)),
            pl.BlockSpec((WBLK, 1),
                         lambda g: (jnp.minimum(g, B2 // WBLK - 1), 0)),
            pl.BlockSpec((B, 1), lambda g: (0, 0)),
            pl.BlockSpec((B, 1), lambda g: (0, 0)),
        ],
        out_shape=[
            jax.ShapeDtypeStruct((NUM_NODES, DE), _f32),
            jax.ShapeDtypeStruct((NUM_NODES, DMAIL), _f32),
            jax.ShapeDtypeStruct((NUM_NODES,), _f32),
            jax.ShapeDtypeStruct((NUM_NODES,), _f32),
            jax.ShapeDtypeStruct((B2, DE), _f32),
            jax.ShapeDtypeStruct((B2, DMAIL), _f32),
            jax.ShapeDtypeStruct((B2, 1), _i32),
            jax.ShapeDtypeStruct((B, 1), _f32),
            jax.ShapeDtypeStruct((B, 1), _f32),
        ],
        scratch_shapes=[
            pltpu.VMEM((B3, DE), _f32),
            pltpu.VMEM((B3, DE), _f32),
            pltpu.SemaphoreType.DMA,
            pltpu.SemaphoreType.DMA,
            pltpu.SemaphoreType.DMA,
            pltpu.SemaphoreType.DMA,
        ],
        name="atlas_k02_fused",
    )(mem, mail, mem_ts, mail_ts, gmem, gmail, gmem_ts, gmail_ts,
      root_ts, edge_feat, dstB, dstC,
      w_ih, bsum, w_hh, time_w, time_b, tl_w, tl_b, ln_g, ln_b,
      ep_srcT, ep_src_b, ep_dstT, ep_dst_b, ep_out_w, ep_out_b)


def _k3_body(dstn, premask, norm8k, gmail_ts, mailrow, root_ts,
             memb, mailb, memtsb, mailtsb,
             out_mem, out_mail, out_mem_ts, out_mail_ts,
             idx_v, pm_v, pmk_v, nrm_v, m2d_v, ts_v, rts_v, sem, rsem):
    del memb, mailb, memtsb, mailtsb
    wid = lax.axis_index("s") * 2 + lax.axis_index("c")
    per_tile = B2 // NW
    base = wid * per_tile
    for j in range(per_tile // CHUNK):
        off = base + j * CHUNK
        pltpu.sync_copy(dstn.at[pl.ds(off, CHUNK)], idx_v)
        pltpu.sync_copy(premask.at[pl.ds(off, CHUNK)], pm_v.at[pl.ds(0, CHUNK)])
        pltpu.sync_copy(premask.at[pl.ds(off, CHUNK)], pmk_v)
        pltpu.sync_copy(norm8k.at[pl.ds(off, CHUNK)], nrm_v)
        pltpu.sync_copy(gmail_ts.at[pl.ds(off, CHUNK)], ts_v)
        pltpu.sync_copy(mailrow.at[pl.ds(off, CHUNK), :], m2d_v)
        pltpu.sync_copy(root_ts.at[pl.ds(off % B, CHUNK)], rts_v)
        a = pltpu.async_copy(nrm_v, out_mem.at[idx_v], sem)
        b = pltpu.async_copy(ts_v, out_mem_ts.at[idx_v], sem)
        c = pltpu.async_copy(
            rts_v, out_mail_ts.at[plsc.Indices(pmk_v, ignored_value=-1)], sem)
        a.wait()
        b.wait()
        c.wait()

        def fire(r, _):
            idx = pm_v[pl.ds(r, 16)][0]

            @pl.when(idx >= 0)
            def _():
                pltpu.make_async_copy(
                    m2d_v.at[pl.ds(r, 1), pl.ds(0, DE)],
                    out_mail.at[pl.ds(idx, 1), pl.ds(0, DE)],
                    rsem,
                ).start()
                pltpu.make_async_copy(
                    m2d_v.at[pl.ds(r, 1), pl.ds(DE, DMAIL - DE)],
                    out_mail.at[pl.ds(idx, 1), pl.ds(DE, DMAIL - DE)],
                    rsem,
                ).start()
            return 0

        lax.fori_loop(0, CHUNK, fire, 0)

        def drain(r, _):
            idx = pm_v[pl.ds(r, 16)][0]

            @pl.when(idx >= 0)
            def _():
                pltpu.make_async_copy(
                    m2d_v.at[pl.ds(r, 1), pl.ds(0, DE)],
                    out_mail.at[pl.ds(idx, 1), pl.ds(0, DE)],
                    rsem,
                ).wait()
                pltpu.make_async_copy(
                    m2d_v.at[pl.ds(r, 1), pl.ds(DE, DMAIL - DE)],
                    out_mail.at[pl.ds(idx, 1), pl.ds(DE, DMAIL - DE)],
                    rsem,
                ).wait()
            return 0

        lax.fori_loop(0, CHUNK, drain, 0)


def _k3(dstn, premask, norm8k, gmail_ts, mailrow, root_ts,
        memb, mailb, memtsb, mailtsb):
    mesh = plsc.VectorSubcoreMesh(core_axis_name="c", subcore_axis_name="s")
    call = _mpmd._mpmd_map(
        [(mesh, _k3_body)],
        (
            jax.ShapeDtypeStruct((NUM_NODES, DE), _f32),
            jax.ShapeDtypeStruct((NUM_NODES, DMAIL), _f32),
            jax.ShapeDtypeStruct((NUM_NODES,), _f32),
            jax.ShapeDtypeStruct((NUM_NODES,), _f32),
        ),
        input_output_aliases={6: 0, 7: 1, 8: 2, 9: 3},
        scratch_types=(
            pltpu.VMEM((CHUNK,), _i32),
            pltpu.VMEM((CHUNK + 16,), _i32),
            pltpu.VMEM((CHUNK,), _i32),
            pltpu.VMEM((CHUNK, DE), _f32),
            pltpu.VMEM((CHUNK, DMAIL), _f32),
            pltpu.VMEM((CHUNK,), _f32),
            pltpu.VMEM((CHUNK,), _f32),
            pltpu.SemaphoreType.DMA,
            pltpu.SemaphoreType.DMA,
        ),
        name="atlas_k3_scatter",
    )
    return call(dstn, premask, norm8k, gmail_ts, mailrow, root_ts,
                memb, mailb, memtsb, mailtsb)


def kernel(dst_nodes, root_ts, root_edge_feat, memory, memory_ts, mail, mail_ts,
           W_ih, b_ih, W_hh, b_hh, time_w, time_b, tl_W, tl_b, ln_g, ln_b,
           ep_src_W, ep_src_b, ep_dst_W, ep_dst_b, ep_out_W, ep_out_b):
    dstn = dst_nodes.astype(_i32)
    dstB = dstn[:B2].reshape(B2, 1)
    dstC = dstn[:B2].reshape(B2 // JBLK, 1, JBLK)

    gmem, gmail, gmem_ts, gmail_ts = _k1(
        memory, mail, memory_ts, mail_ts, dstn)

    weights = (
        W_ih.T, (b_ih + b_hh).reshape(1, DE), W_hh.T,
        time_w.reshape(1, DT), time_b.reshape(1, DT),
        tl_W[:, 0].reshape(1, DE), tl_b.reshape(1, DE),
        ln_g.reshape(1, DE), ln_b.reshape(1, DE),
        ep_src_W.T, ep_src_b.reshape(1, DE),
        ep_dst_W.T, ep_dst_b.reshape(1, DE),
        ep_out_W.T, ep_out_b.reshape(1, 1),
    )
    (memb, mailb, memtsb, mailtsb, norm8k, mailrow, loser2,
     pos_scores, neg_scores) = _k02(
        memory, mail, memory_ts, mail_ts, gmem, gmail, gmem_ts, gmail_ts,
        root_ts, root_edge_feat, dstB, dstC, weights)

    premask = jnp.where(loser2.reshape(B2) == 0, dstn[:B2], -1).astype(_i32)

    new_memory, new_mail, new_memory_ts, new_mail_ts = _k3(
        dstn[:B2], premask, norm8k, gmail_ts[:B2], mailrow, root_ts,
        memb, mailb, memtsb, mailtsb)

    return (pos_scores, neg_scores, new_memory, new_memory_ts,
            new_mail, new_mail_ts)


# speedup vs baseline: 12.6458x; 12.6458x over previous
import jax
import jax.numpy as jnp
from jax import lax
from jax.experimental import pallas as pl
from jax.experimental.pallas import tpu as pltpu
from jax.experimental.pallas import tpu_sc as plsc
from jax._src.pallas import mpmd as _mpmd

NUM_NODES = 100000
B = 4096
B2 = 2 * B
B3 = 3 * B
DE = 128
DMAIL = 144
DT = 100
NW = 32
CHUNK = 128
RN = 1024
RC = 4000
NSTEPS = NUM_NODES // RC
WBLK = 512
JBLK = 2048

_f32 = jnp.float32
_i32 = jnp.int32


def _k1_body(mem_hbm, mail_hbm, mem_ts_hbm, mail_ts_hbm, idx_hbm,
             gmem, gmail, gmem_ts, gmail_ts,
             idx_v, idxp_v, rows_a, mrow_v, ts_v, ts2_v, sem, rsem):
    wid = lax.axis_index("s") * 2 + lax.axis_index("c")
    per_tile = B3 // NW
    base = wid * per_tile
    for j in range(per_tile // CHUNK):
        off = base + j * CHUNK
        pltpu.sync_copy(idx_hbm.at[pl.ds(off, CHUNK)], idx_v)
        pltpu.sync_copy(idx_hbm.at[pl.ds(off, CHUNK)],
                        idxp_v.at[pl.ds(0, CHUNK)])
        a = pltpu.async_copy(mem_hbm.at[idx_v], rows_a, sem)
        c = pltpu.async_copy(mem_ts_hbm.at[idx_v], ts_v, sem)
        d = pltpu.async_copy(mail_ts_hbm.at[idx_v], ts2_v, sem)

        def fire(r, _):
            idx = idxp_v[pl.ds(r, 16)][0]
            pltpu.make_async_copy(
                mail_hbm.at[pl.ds(idx, 1), pl.ds(0, DE)],
                mrow_v.at[pl.ds(r, 1), pl.ds(0, DE)],
                rsem,
            ).start()
            pltpu.make_async_copy(
                mail_hbm.at[pl.ds(idx, 1), pl.ds(DE, DMAIL - DE)],
                mrow_v.at[pl.ds(r, 1), pl.ds(DE, DMAIL - DE)],
                rsem,
            ).start()
            return 0

        lax.fori_loop(0, CHUNK, fire, 0)

        def drain(r, _):
            idx = idxp_v[pl.ds(r, 16)][0]
            pltpu.make_async_copy(
                mail_hbm.at[pl.ds(idx, 1), pl.ds(0, DE)],
                mrow_v.at[pl.ds(r, 1), pl.ds(0, DE)],
                rsem,
            ).wait()
            pltpu.make_async_copy(
                mail_hbm.at[pl.ds(idx, 1), pl.ds(DE, DMAIL - DE)],
                mrow_v.at[pl.ds(r, 1), pl.ds(DE, DMAIL - DE)],
                rsem,
            ).wait()
            return 0

        lax.fori_loop(0, CHUNK, drain, 0)
        a.wait()
        c.wait()
        d.wait()
        pltpu.sync_copy(rows_a, gmem.at[pl.ds(off, CHUNK)])
        pltpu.sync_copy(mrow_v, gmail.at[pl.ds(off, CHUNK), :])
        pltpu.sync_copy(ts_v, gmem_ts.at[pl.ds(off, CHUNK)])
        pltpu.sync_copy(ts2_v, gmail_ts.at[pl.ds(off, CHUNK)])


_k1 = pl.kernel(
    _k1_body,
    out_type=(
        jax.ShapeDtypeStruct((B3, DE), _f32),
        jax.ShapeDtypeStruct((B3, DMAIL), _f32),
        jax.ShapeDtypeStruct((B3,), _f32),
        jax.ShapeDtypeStruct((B3,), _f32),
    ),
    mesh=plsc.VectorSubcoreMesh(core_axis_name="c", subcore_axis_name="s"),
    scratch_types=(
        pltpu.VMEM((CHUNK,), _i32),
        pltpu.VMEM((CHUNK + 16,), _i32),
        pltpu.VMEM((CHUNK, DE), _f32),
        pltpu.VMEM((CHUNK, DMAIL), _f32),
        pltpu.VMEM((CHUNK,), _f32),
        pltpu.VMEM((CHUNK,), _f32),
        pltpu.SemaphoreType.DMA,
        pltpu.SemaphoreType.DMA,
    ),
    name="atlas_k1_gather",
)


def _kcopy_body(mem_in, mail_in, mem_ts_in, mail_ts_in,
                out_mem, out_mail, out_mem_ts, out_mail_ts):
    g = pl.program_id(0)
    out_mem[...] = mem_in[...]
    out_mail[...] = mail_in[...]

    @pl.when(g == 0)
    def _ts():
        out_mem_ts[...] = mem_ts_in[...]
        out_mail_ts[...] = mail_ts_in[...]


def _kcopy(mem, mail, mem_ts, mail_ts):
    return pl.pallas_call(
        _kcopy_body,
        grid=(NSTEPS,),
        in_specs=[
            pl.BlockSpec((RC, DE), lambda g: (g, 0)),
            pl.BlockSpec((RC, DMAIL), lambda g: (g, 0)),
            pl.BlockSpec((NUM_NODES,), lambda g: (0,)),
            pl.BlockSpec((NUM_NODES,), lambda g: (0,)),
        ],
        out_specs=[
            pl.BlockSpec((RC, DE), lambda g: (g, 0)),
            pl.BlockSpec((RC, DMAIL), lambda g: (g, 0)),
            pl.BlockSpec((NUM_NODES,), lambda g: (0,)),
            pl.BlockSpec((NUM_NODES,), lambda g: (0,)),
        ],
        out_shape=[
            jax.ShapeDtypeStruct((NUM_NODES, DE), _f32),
            jax.ShapeDtypeStruct((NUM_NODES, DMAIL), _f32),
            jax.ShapeDtypeStruct((NUM_NODES,), _f32),
            jax.ShapeDtypeStruct((NUM_NODES,), _f32),
        ],
        name="atlas_kcopy",
    )(mem, mail, mem_ts, mail_ts)


def _k02_body(gmem, gmail, gmem_ts, gmail_ts, root_ts, edge_feat,
              dstB_ref, dstC_ref,
              w_ih, bsum, w_hh, time_w, time_b, tl_w, tl_b, ln_g, ln_b,
              ep_srcT, ep_src_b, ep_dstT, ep_dst_b, ep_out_w, ep_out_b,
              norm8k, mailrow, loser_out, pos_out, neg_out,
              norm_scr, proj_scr):
    g = pl.program_id(0)

    @pl.when(g < B3 // RN)
    def _dense():
        prev_mem = gmem[...]
        m = gmail[...]
        prev_ts = gmem_ts[...]
        m_ts = gmail_ts[...]
        dt = m_ts - prev_ts
        tf = jnp.cos(dt[:, None] * time_w[...] + time_b[...])
        x = jnp.concatenate([m, tf], axis=1)
        up = jnp.tanh(
            jnp.dot(x, w_ih[...], preferred_element_type=_f32)
            + jnp.dot(prev_mem, w_hh[...], preferred_element_type=_f32)
            + bsum[...]
        )
        mu = jnp.mean(up, axis=1, keepdims=True)
        var = jnp.mean((up - mu) ** 2, axis=1, keepdims=True)
        norm = (up - mu) / jnp.sqrt(var + 1e-5) * ln_g[...] + ln_b[...]
        times = root_ts[...]
        tidiff = (times - m_ts) / (times + 1.0)
        proj = norm * (1.0 + tidiff[:, None] * tl_w[...] + tl_b[...])
        norm_scr[pl.ds(g * RN, RN), :] = norm
        proj_scr[pl.ds(g * RN, RN), :] = proj

        @pl.when(g < B2 // RN)
        def _():
            norm8k[...] = norm

    @pl.when((g >= 12) & (g < 20))
    def _mailrow():
        k = g - 12
        rows0 = k * RN
        src_off = jnp.where(rows0 < B, rows0 + B, rows0 - B)
        me = norm_scr[pl.ds(src_off, RN), :]
        mailrow[...] = jnp.concatenate([me, edge_feat[...]], axis=1)

    @pl.when((g >= 20) & (g < 24))
    def _scores():
        q = g - 20
        r0 = q * RN
        src = proj_scr[pl.ds(r0, RN), :]
        dst = proj_scr[pl.ds(B + r0, RN), :]
        neg = proj_scr[pl.ds(2 * B + r0, RN), :]
        sx = jnp.dot(src, ep_srcT[...], preferred_element_type=_f32) + ep_src_b[...]
        hp = jnp.maximum(sx + jnp.dot(dst, ep_dstT[...], preferred_element_type=_f32) + ep_dst_b[...], 0.0)
        hn = jnp.maximum(sx + jnp.dot(neg, ep_dstT[...], preferred_element_type=_f32) + ep_dst_b[...], 0.0)
        pos_out[pl.ds(r0, RN), :] = jnp.dot(hp, ep_out_w[...], preferred_element_type=_f32) + ep_out_b[...]
        neg_out[pl.ds(r0, RN), :] = jnp.dot(hn, ep_out_w[...], preferred_element_type=_f32) + ep_out_b[...]

    @pl.when(g < B2 // WBLK)
    def _loser():
        i0 = g * WBLK
        pos_i = dstB_ref[pl.ds(i0, WBLK), :]
        cd = i0 // JBLK

        pos_jd = dstC_ref[pl.ds(cd, 1), 0, :]
        jd = cd * JBLK + lax.broadcasted_iota(_i32, (WBLK, JBLK), 1)
        icol = i0 + lax.broadcasted_iota(_i32, (WBLK, JBLK), 0)
        acc0 = jnp.any(
            (pos_i == pos_jd) & (jd > icol), axis=1, keepdims=True
        ).astype(_i32)

        def body(c, acc):
            pos_j = dstC_ref[pl.ds(c, 1), 0, :]
            hit = jnp.any(pos_i == pos_j, axis=1, keepdims=True).astype(_i32)
            return jnp.maximum(acc, hit)

        acc = lax.fori_loop(cd + 1, B2 // JBLK, body, acc0)
        loser_out[...] = acc


def _k02(gmem, gmail, gmem_ts, gmail_ts,
         root_ts, edge_feat, dstB, dstC, weights):
    (w_ih, bsum, w_hh, time_w, time_b, tl_w, tl_b, ln_g, ln_b,
     ep_srcT, ep_src_b, ep_dstT, ep_dst_b, ep_out_w, ep_out_b) = weights
    full = lambda shape: pl.BlockSpec(shape, lambda g: tuple(0 for _ in shape))
    dense_i = lambda g: (jnp.minimum(g, B3 // RN - 1), 0)
    dense_i1 = lambda g: (jnp.minimum(g, B3 // RN - 1),)
    return pl.pallas_call(
        _k02_body,
        grid=(24,),
        in_specs=[
            pl.BlockSpec((RN, DE), dense_i),
            pl.BlockSpec((RN, DMAIL), dense_i),
            pl.BlockSpec((RN,), dense_i1),
            pl.BlockSpec((RN,), dense_i1),
            pl.BlockSpec((RN,), lambda g: (g % 4,)),
            pl.BlockSpec((RN, 16), lambda g: (jnp.clip(g - 12, 0, 7) % 4, 0)),
            pl.BlockSpec((B2, 1), lambda g: (0, 0)),
            pl.BlockSpec((B2 // JBLK, 1, JBLK), lambda g: (0, 0, 0)),
            full((244, DE)),
            full((1, DE)),
            full((DE, DE)),
            full((1, DT)),
            full((1, DT)),
            full((1, DE)),
            full((1, DE)),
            full((1, DE)),
            full((1, DE)),
            full((DE, DE)),
            full((1, DE)),
            full((DE, DE)),
            full((1, DE)),
            full((DE, 1)),
            full((1, 1)),
        ],
        out_specs=[
            pl.BlockSpec((RN, DE), lambda g: (jnp.minimum(g, B2 // RN - 1), 0)),
            pl.BlockSpec((RN, DMAIL), lambda g: (jnp.clip(g - 12, 0, 7), 0)),
            pl.BlockSpec((WBLK, 1),
                         lambda g: (jnp.minimum(g, B2 // WBLK - 1), 0)),
            pl.BlockSpec((B, 1), lambda g: (0, 0)),
            pl.BlockSpec((B, 1), lambda g: (0, 0)),
        ],
        out_shape=[
            jax.ShapeDtypeStruct((B2, DE), _f32),
            jax.ShapeDtypeStruct((B2, DMAIL), _f32),
            jax.ShapeDtypeStruct((B2, 1), _i32),
            jax.ShapeDtypeStruct((B, 1), _f32),
            jax.ShapeDtypeStruct((B, 1), _f32),
        ],
        scratch_shapes=[
            pltpu.VMEM((B3, DE), _f32),
            pltpu.VMEM((B3, DE), _f32),
        ],
        name="atlas_k02_fused",
    )(gmem, gmail, gmem_ts, gmail_ts,
      root_ts, edge_feat, dstB, dstC,
      w_ih, bsum, w_hh, time_w, time_b, tl_w, tl_b, ln_g, ln_b,
      ep_srcT, ep_src_b, ep_dstT, ep_dst_b, ep_out_w, ep_out_b)


def _k3_body(dstn, premask, norm8k, gmail_ts, mailrow, root_ts,
             memb, mailb, memtsb, mailtsb,
             out_mem, out_mail, out_mem_ts, out_mail_ts,
             idx_v, pm_v, pmk_v, nrm_v, m2d_v, ts_v, rts_v, sem, rsem):
    del memb, mailb, memtsb, mailtsb
    wid = lax.axis_index("s") * 2 + lax.axis_index("c")
    per_tile = B2 // NW
    base = wid * per_tile
    for j in range(per_tile // CHUNK):
        off = base + j * CHUNK
        pltpu.sync_copy(dstn.at[pl.ds(off, CHUNK)], idx_v)
        pltpu.sync_copy(premask.at[pl.ds(off, CHUNK)], pm_v.at[pl.ds(0, CHUNK)])
        pltpu.sync_copy(premask.at[pl.ds(off, CHUNK)], pmk_v)
        pltpu.sync_copy(norm8k.at[pl.ds(off, CHUNK)], nrm_v)
        pltpu.sync_copy(gmail_ts.at[pl.ds(off, CHUNK)], ts_v)
        pltpu.sync_copy(mailrow.at[pl.ds(off, CHUNK), :], m2d_v)
        pltpu.sync_copy(root_ts.at[pl.ds(off % B, CHUNK)], rts_v)
        a = pltpu.async_copy(nrm_v, out_mem.at[idx_v], sem)
        b = pltpu.async_copy(ts_v, out_mem_ts.at[idx_v], sem)
        c = pltpu.async_copy(
            rts_v, out_mail_ts.at[plsc.Indices(pmk_v, ignored_value=-1)], sem)
        a.wait()
        b.wait()
        c.wait()

        def fire(r, _):
            idx = pm_v[pl.ds(r, 16)][0]

            @pl.when(idx >= 0)
            def _():
                pltpu.make_async_copy(
                    m2d_v.at[pl.ds(r, 1), pl.ds(0, DE)],
                    out_mail.at[pl.ds(idx, 1), pl.ds(0, DE)],
                    rsem,
                ).start()
                pltpu.make_async_copy(
                    m2d_v.at[pl.ds(r, 1), pl.ds(DE, DMAIL - DE)],
                    out_mail.at[pl.ds(idx, 1), pl.ds(DE, DMAIL - DE)],
                    rsem,
                ).start()
            return 0

        lax.fori_loop(0, CHUNK, fire, 0)

        def drain(r, _):
            idx = pm_v[pl.ds(r, 16)][0]

            @pl.when(idx >= 0)
            def _():
                pltpu.make_async_copy(
                    m2d_v.at[pl.ds(r, 1), pl.ds(0, DE)],
                    out_mail.at[pl.ds(idx, 1), pl.ds(0, DE)],
                    rsem,
                ).wait()
                pltpu.make_async_copy(
                    m2d_v.at[pl.ds(r, 1), pl.ds(DE, DMAIL - DE)],
                    out_mail.at[pl.ds(idx, 1), pl.ds(DE, DMAIL - DE)],
                    rsem,
                ).wait()
            return 0

        lax.fori_loop(0, CHUNK, drain, 0)


def _k3(dstn, premask, norm8k, gmail_ts, mailrow, root_ts,
        memb, mailb, memtsb, mailtsb):
    mesh = plsc.VectorSubcoreMesh(core_axis_name="c", subcore_axis_name="s")
    call = _mpmd._mpmd_map(
        [(mesh, _k3_body)],
        (
            jax.ShapeDtypeStruct((NUM_NODES, DE), _f32),
            jax.ShapeDtypeStruct((NUM_NODES, DMAIL), _f32),
            jax.ShapeDtypeStruct((NUM_NODES,), _f32),
            jax.ShapeDtypeStruct((NUM_NODES,), _f32),
        ),
        input_output_aliases={6: 0, 7: 1, 8: 2, 9: 3},
        scratch_types=(
            pltpu.VMEM((CHUNK,), _i32),
            pltpu.VMEM((CHUNK + 16,), _i32),
            pltpu.VMEM((CHUNK,), _i32),
            pltpu.VMEM((CHUNK, DE), _f32),
            pltpu.VMEM((CHUNK, DMAIL), _f32),
            pltpu.VMEM((CHUNK,), _f32),
            pltpu.VMEM((CHUNK,), _f32),
            pltpu.SemaphoreType.DMA,
            pltpu.SemaphoreType.DMA,
        ),
        name="atlas_k3_scatter",
    )
    return call(dstn, premask, norm8k, gmail_ts, mailrow, root_ts,
                memb, mailb, memtsb, mailtsb)


def kernel(dst_nodes, root_ts, root_edge_feat, memory, memory_ts, mail, mail_ts,
           W_ih, b_ih, W_hh, b_hh, time_w, time_b, tl_W, tl_b, ln_g, ln_b,
           ep_src_W, ep_src_b, ep_dst_W, ep_dst_b, ep_out_W, ep_out_b):
    dstn = dst_nodes.astype(_i32)
    dstB = dstn[:B2].reshape(B2, 1)
    dstC = dstn[:B2].reshape(B2 // JBLK, 1, JBLK)

    gmem, gmail, gmem_ts, gmail_ts = _k1(
        memory, mail, memory_ts, mail_ts, dstn)
    memb, mailb, memtsb, mailtsb = _kcopy(memory, mail, memory_ts, mail_ts)

    weights = (
        W_ih.T, (b_ih + b_hh).reshape(1, DE), W_hh.T,
        time_w.reshape(1, DT), time_b.reshape(1, DT),
        tl_W[:, 0].reshape(1, DE), tl_b.reshape(1, DE),
        ln_g.reshape(1, DE), ln_b.reshape(1, DE),
        ep_src_W.T, ep_src_b.reshape(1, DE),
        ep_dst_W.T, ep_dst_b.reshape(1, DE),
        ep_out_W.T, ep_out_b.reshape(1, 1),
    )
    (norm8k, mailrow, loser2, pos_scores, neg_scores) = _k02(
        gmem, gmail, gmem_ts, gmail_ts,
        root_ts, root_edge_feat, dstB, dstC, weights)

    premask = jnp.where(loser2.reshape(B2) == 0, dstn[:B2], -1).astype(_i32)

    new_memory, new_mail, new_memory_ts, new_mail_ts = _k3(
        dstn[:B2], premask, norm8k, gmail_ts[:B2], mailrow, root_ts,
        memb, mailb, memtsb, mailtsb)

    return (pos_scores, neg_scores, new_memory, new_memory_ts,
            new_mail, new_mail_ts)


# speedup vs baseline: 14.2873x; 1.1298x over previous
import jax
import jax.numpy as jnp
from jax import lax
from jax.experimental import pallas as pl
from jax.experimental.pallas import tpu as pltpu
from jax.experimental.pallas import tpu_sc as plsc
from jax._src.pallas import mpmd as _mpmd

NUM_NODES = 100000
B = 4096
B2 = 2 * B
B3 = 3 * B
DE = 128
DMAIL = 144
DT = 100
NW = 32
CHUNK = 128
RN = 1024
RC = 4000
NSTEPS = NUM_NODES // RC
WBLK = 512
JBLK = 2048

_f32 = jnp.float32
_i32 = jnp.int32


def _k1_body(mem_hbm, mail_hbm, mem_ts_hbm, mail_ts_hbm, idx_hbm,
             gmem, gmail, gmem_ts, gmail_ts,
             idx_v, idxp_v, rows_a, mrow_v, ts_v, ts2_v, sem, rsem):
    wid = lax.axis_index("s") * 2 + lax.axis_index("c")
    per_tile = B3 // NW
    base = wid * per_tile
    for j in range(per_tile // CHUNK):
        off = base + j * CHUNK
        pltpu.sync_copy(idx_hbm.at[pl.ds(off, CHUNK)], idx_v)
        pltpu.sync_copy(idx_hbm.at[pl.ds(off, CHUNK)],
                        idxp_v.at[pl.ds(0, CHUNK)])
        a = pltpu.async_copy(mem_hbm.at[idx_v], rows_a, sem)
        c = pltpu.async_copy(mem_ts_hbm.at[idx_v], ts_v, sem)
        d = pltpu.async_copy(mail_ts_hbm.at[idx_v], ts2_v, sem)

        def fire(r, _):
            idx = idxp_v[pl.ds(r, 16)][0]
            pltpu.make_async_copy(
                mail_hbm.at[pl.ds(idx, 1), pl.ds(0, DE)],
                mrow_v.at[pl.ds(r, 1), pl.ds(0, DE)],
                rsem,
            ).start()
            pltpu.make_async_copy(
                mail_hbm.at[pl.ds(idx, 1), pl.ds(DE, DMAIL - DE)],
                mrow_v.at[pl.ds(r, 1), pl.ds(DE, DMAIL - DE)],
                rsem,
            ).start()
            return 0

        lax.fori_loop(0, CHUNK, fire, 0)

        def drain(r, _):
            idx = idxp_v[pl.ds(r, 16)][0]
            pltpu.make_async_copy(
                mail_hbm.at[pl.ds(idx, 1), pl.ds(0, DE)],
                mrow_v.at[pl.ds(r, 1), pl.ds(0, DE)],
                rsem,
            ).wait()
            pltpu.make_async_copy(
                mail_hbm.at[pl.ds(idx, 1), pl.ds(DE, DMAIL - DE)],
                mrow_v.at[pl.ds(r, 1), pl.ds(DE, DMAIL - DE)],
                rsem,
            ).wait()
            return 0

        lax.fori_loop(0, CHUNK, drain, 0)
        a.wait()
        c.wait()
        d.wait()
        pltpu.sync_copy(rows_a, gmem.at[pl.ds(off, CHUNK)])
        pltpu.sync_copy(mrow_v, gmail.at[pl.ds(off, CHUNK), :])
        pltpu.sync_copy(ts_v, gmem_ts.at[pl.ds(off, CHUNK)])
        pltpu.sync_copy(ts2_v, gmail_ts.at[pl.ds(off, CHUNK)])


_k1 = pl.kernel(
    _k1_body,
    out_type=(
        jax.ShapeDtypeStruct((B3, DE), _f32),
        jax.ShapeDtypeStruct((B3, DMAIL), _f32),
        jax.ShapeDtypeStruct((B3,), _f32),
        jax.ShapeDtypeStruct((B3,), _f32),
    ),
    mesh=plsc.VectorSubcoreMesh(core_axis_name="c", subcore_axis_name="s"),
    scratch_types=(
        pltpu.VMEM((CHUNK,), _i32),
        pltpu.VMEM((CHUNK + 16,), _i32),
        pltpu.VMEM((CHUNK, DE), _f32),
        pltpu.VMEM((CHUNK, DMAIL), _f32),
        pltpu.VMEM((CHUNK,), _f32),
        pltpu.VMEM((CHUNK,), _f32),
        pltpu.SemaphoreType.DMA,
        pltpu.SemaphoreType.DMA,
    ),
    name="atlas_k1_gather",
)


def _k02_body(mem_in, mail_in, mem_ts_in, mail_ts_in,
              gmem, gmail, gmem_ts, gmail_ts, root_ts, edge_feat,
              dstB_ref, dstC_ref,
              w_ih, bsum, w_hh, time_w, time_b, tl_w, tl_b, ln_g, ln_b,
              ep_srcT, ep_src_b, ep_dstT, ep_dst_b, ep_out_w, ep_out_b,
              out_mem, out_mail, out_mem_ts, out_mail_ts,
              norm8k, mailrow, loser_out, pos_out, neg_out,
              norm_scr, proj_scr):
    g = pl.program_id(0)
    out_mem[...] = mem_in[...]
    out_mail[...] = mail_in[...]

    @pl.when(g == 0)
    def _ts():
        out_mem_ts[...] = mem_ts_in[...]
        out_mail_ts[...] = mail_ts_in[...]

    @pl.when(g < B3 // RN)
    def _dense():
        prev_mem = gmem[...]
        m = gmail[...]
        prev_ts = gmem_ts[...]
        m_ts = gmail_ts[...]
        dt = m_ts - prev_ts
        tf = jnp.cos(dt[:, None] * time_w[...] + time_b[...])
        x = jnp.concatenate([m, tf], axis=1)
        up = jnp.tanh(
            jnp.dot(x, w_ih[...], preferred_element_type=_f32)
            + jnp.dot(prev_mem, w_hh[...], preferred_element_type=_f32)
            + bsum[...]
        )
        mu = jnp.mean(up, axis=1, keepdims=True)
        var = jnp.mean((up - mu) ** 2, axis=1, keepdims=True)
        norm = (up - mu) / jnp.sqrt(var + 1e-5) * ln_g[...] + ln_b[...]
        times = root_ts[...]
        tidiff = (times - m_ts) / (times + 1.0)
        proj = norm * (1.0 + tidiff[:, None] * tl_w[...] + tl_b[...])
        norm_scr[pl.ds(g * RN, RN), :] = norm
        proj_scr[pl.ds(g * RN, RN), :] = proj

        @pl.when(g < B2 // RN)
        def _():
            norm8k[...] = norm

    @pl.when((g >= 12) & (g < 20))
    def _mailrow():
        k = g - 12
        rows0 = k * RN
        src_off = jnp.where(rows0 < B, rows0 + B, rows0 - B)
        me = norm_scr[pl.ds(src_off, RN), :]
        mailrow[...] = jnp.concatenate([me, edge_feat[...]], axis=1)

    @pl.when((g >= 20) & (g < 24))
    def _scores():
        q = g - 20
        r0 = q * RN
        src = proj_scr[pl.ds(r0, RN), :]
        dst = proj_scr[pl.ds(B + r0, RN), :]
        neg = proj_scr[pl.ds(2 * B + r0, RN), :]
        sx = jnp.dot(src, ep_srcT[...], preferred_element_type=_f32) + ep_src_b[...]
        hp = jnp.maximum(sx + jnp.dot(dst, ep_dstT[...], preferred_element_type=_f32) + ep_dst_b[...], 0.0)
        hn = jnp.maximum(sx + jnp.dot(neg, ep_dstT[...], preferred_element_type=_f32) + ep_dst_b[...], 0.0)
        pos_out[pl.ds(r0, RN), :] = jnp.dot(hp, ep_out_w[...], preferred_element_type=_f32) + ep_out_b[...]
        neg_out[pl.ds(r0, RN), :] = jnp.dot(hn, ep_out_w[...], preferred_element_type=_f32) + ep_out_b[...]

    @pl.when(g < B2 // WBLK)
    def _loser():
        i0 = g * WBLK
        pos_i = dstB_ref[pl.ds(i0, WBLK), :]
        cd = i0 // JBLK

        pos_jd = dstC_ref[pl.ds(cd, 1), 0, :]
        jd = cd * JBLK + lax.broadcasted_iota(_i32, (WBLK, JBLK), 1)
        icol = i0 + lax.broadcasted_iota(_i32, (WBLK, JBLK), 0)
        acc0 = jnp.any(
            (pos_i == pos_jd) & (jd > icol), axis=1, keepdims=True
        ).astype(_i32)

        def body(c, acc):
            pos_j = dstC_ref[pl.ds(c, 1), 0, :]
            hit = jnp.any(pos_i == pos_j, axis=1, keepdims=True).astype(_i32)
            return jnp.maximum(acc, hit)

        acc = lax.fori_loop(cd + 1, B2 // JBLK, body, acc0)
        loser_out[...] = acc


def _k02(mem, mail, mem_ts, mail_ts, gmem, gmail, gmem_ts, gmail_ts,
         root_ts, edge_feat, dstB, dstC, weights):
    (w_ih, bsum, w_hh, time_w, time_b, tl_w, tl_b, ln_g, ln_b,
     ep_srcT, ep_src_b, ep_dstT, ep_dst_b, ep_out_w, ep_out_b) = weights
    full = lambda shape: pl.BlockSpec(shape, lambda g: tuple(0 for _ in shape))
    dense_i = lambda g: (jnp.minimum(g, B3 // RN - 1), 0)
    dense_i1 = lambda g: (jnp.minimum(g, B3 // RN - 1),)
    return pl.pallas_call(
        _k02_body,
        grid=(NSTEPS,),
        in_specs=[
            pl.BlockSpec((RC, DE), lambda g: (g, 0)),
            pl.BlockSpec((RC, DMAIL), lambda g: (g, 0)),
            pl.BlockSpec((NUM_NODES,), lambda g: (0,)),
            pl.BlockSpec((NUM_NODES,), lambda g: (0,)),
            pl.BlockSpec((RN, DE), dense_i),
            pl.BlockSpec((RN, DMAIL), dense_i),
            pl.BlockSpec((RN,), dense_i1),
            pl.BlockSpec((RN,), dense_i1),
            pl.BlockSpec((RN,), lambda g: (g % 4,)),
            pl.BlockSpec((RN, 16), lambda g: (jnp.clip(g - 12, 0, 7) % 4, 0)),
            pl.BlockSpec((B2, 1), lambda g: (0, 0)),
            pl.BlockSpec((B2 // JBLK, 1, JBLK), lambda g: (0, 0, 0)),
            full((244, DE)),
            full((1, DE)),
            full((DE, DE)),
            full((1, DT)),
            full((1, DT)),
            full((1, DE)),
            full((1, DE)),
            full((1, DE)),
            full((1, DE)),
            full((DE, DE)),
            full((1, DE)),
            full((DE, DE)),
            full((1, DE)),
            full((DE, 1)),
            full((1, 1)),
        ],
        out_specs=[
            pl.BlockSpec((RC, DE), lambda g: (g, 0)),
            pl.BlockSpec((RC, DMAIL), lambda g: (g, 0)),
            pl.BlockSpec((NUM_NODES,), lambda g: (0,)),
            pl.BlockSpec((NUM_NODES,), lambda g: (0,)),
            pl.BlockSpec((RN, DE), lambda g: (jnp.minimum(g, B2 // RN - 1), 0)),
            pl.BlockSpec((RN, DMAIL), lambda g: (jnp.clip(g - 12, 0, 7), 0)),
            pl.BlockSpec((WBLK, 1),
                         lambda g: (jnp.minimum(g, B2 // WBLK - 1), 0)),
            pl.BlockSpec((B, 1), lambda g: (0, 0)),
            pl.BlockSpec((B, 1), lambda g: (0, 0)),
        ],
        out_shape=[
            jax.ShapeDtypeStruct((NUM_NODES, DE), _f32),
            jax.ShapeDtypeStruct((NUM_NODES, DMAIL), _f32),
            jax.ShapeDtypeStruct((NUM_NODES,), _f32),
            jax.ShapeDtypeStruct((NUM_NODES,), _f32),
            jax.ShapeDtypeStruct((B2, DE), _f32),
            jax.ShapeDtypeStruct((B2, DMAIL), _f32),
            jax.ShapeDtypeStruct((B2, 1), _i32),
            jax.ShapeDtypeStruct((B, 1), _f32),
            jax.ShapeDtypeStruct((B, 1), _f32),
        ],
        scratch_shapes=[
            pltpu.VMEM((B3, DE), _f32),
            pltpu.VMEM((B3, DE), _f32),
        ],
        name="atlas_k02_fused",
    )(mem, mail, mem_ts, mail_ts, gmem, gmail, gmem_ts, gmail_ts,
      root_ts, edge_feat, dstB, dstC,
      w_ih, bsum, w_hh, time_w, time_b, tl_w, tl_b, ln_g, ln_b,
      ep_srcT, ep_src_b, ep_dstT, ep_dst_b, ep_out_w, ep_out_b)


def _k3_body(dstn, premask, norm8k, gmail_ts, mailrow, root_ts,
             memb, mailb, memtsb, mailtsb,
             out_mem, out_mail, out_mem_ts, out_mail_ts,
             idx_v, pm_v, pmk_v, nrm_v, m2d_v, ts_v, rts_v, sem, rsem):
    del memb, mailb, memtsb, mailtsb
    wid = lax.axis_index("s") * 2 + lax.axis_index("c")
    per_tile = B2 // NW
    base = wid * per_tile
    for j in range(per_tile // CHUNK):
        off = base + j * CHUNK
        l1 = pltpu.async_copy(dstn.at[pl.ds(off, CHUNK)], idx_v, rsem)
        l2 = pltpu.async_copy(premask.at[pl.ds(off, CHUNK)],
                              pm_v.at[pl.ds(0, CHUNK)], rsem)
        l3 = pltpu.async_copy(premask.at[pl.ds(off, CHUNK)], pmk_v, rsem)
        l4 = pltpu.async_copy(norm8k.at[pl.ds(off, CHUNK)], nrm_v, rsem)
        l5 = pltpu.async_copy(gmail_ts.at[pl.ds(off, CHUNK)], ts_v, rsem)
        l6 = pltpu.async_copy(mailrow.at[pl.ds(off, CHUNK), :], m2d_v, rsem)
        l7 = pltpu.async_copy(root_ts.at[pl.ds(off % B, CHUNK)], rts_v, rsem)
        for l in (l1, l2, l3, l4, l5, l6, l7):
            l.wait()
        a = pltpu.async_copy(nrm_v, out_mem.at[idx_v], sem)
        b = pltpu.async_copy(ts_v, out_mem_ts.at[idx_v], sem)
        c = pltpu.async_copy(
            rts_v, out_mail_ts.at[plsc.Indices(pmk_v, ignored_value=-1)], sem)

        def fire(r, _):
            idx = pm_v[pl.ds(r, 16)][0]

            @pl.when(idx >= 0)
            def _():
                pltpu.make_async_copy(
                    m2d_v.at[pl.ds(r, 1), pl.ds(0, DE)],
                    out_mail.at[pl.ds(idx, 1), pl.ds(0, DE)],
                    rsem,
                ).start()
                pltpu.make_async_copy(
                    m2d_v.at[pl.ds(r, 1), pl.ds(DE, DMAIL - DE)],
                    out_mail.at[pl.ds(idx, 1), pl.ds(DE, DMAIL - DE)],
                    rsem,
                ).start()
            return 0

        lax.fori_loop(0, CHUNK, fire, 0)
        a.wait()
        b.wait()
        c.wait()

        def drain(r, _):
            idx = pm_v[pl.ds(r, 16)][0]

            @pl.when(idx >= 0)
            def _():
                pltpu.make_async_copy(
                    m2d_v.at[pl.ds(r, 1), pl.ds(0, DE)],
                    out_mail.at[pl.ds(idx, 1), pl.ds(0, DE)],
                    rsem,
                ).wait()
                pltpu.make_async_copy(
                    m2d_v.at[pl.ds(r, 1), pl.ds(DE, DMAIL - DE)],
                    out_mail.at[pl.ds(idx, 1), pl.ds(DE, DMAIL - DE)],
                    rsem,
                ).wait()
            return 0

        lax.fori_loop(0, CHUNK, drain, 0)


def _k3(dstn, premask, norm8k, gmail_ts, mailrow, root_ts,
        memb, mailb, memtsb, mailtsb):
    mesh = plsc.VectorSubcoreMesh(core_axis_name="c", subcore_axis_name="s")
    call = _mpmd._mpmd_map(
        [(mesh, _k3_body)],
        (
            jax.ShapeDtypeStruct((NUM_NODES, DE), _f32),
            jax.ShapeDtypeStruct((NUM_NODES, DMAIL), _f32),
            jax.ShapeDtypeStruct((NUM_NODES,), _f32),
            jax.ShapeDtypeStruct((NUM_NODES,), _f32),
        ),
        input_output_aliases={6: 0, 7: 1, 8: 2, 9: 3},
        scratch_types=(
            pltpu.VMEM((CHUNK,), _i32),
            pltpu.VMEM((CHUNK + 16,), _i32),
            pltpu.VMEM((CHUNK,), _i32),
            pltpu.VMEM((CHUNK, DE), _f32),
            pltpu.VMEM((CHUNK, DMAIL), _f32),
            pltpu.VMEM((CHUNK,), _f32),
            pltpu.VMEM((CHUNK,), _f32),
            pltpu.SemaphoreType.DMA,
            pltpu.SemaphoreType.DMA,
        ),
        name="atlas_k3_scatter",
    )
    return call(dstn, premask, norm8k, gmail_ts, mailrow, root_ts,
                memb, mailb, memtsb, mailtsb)


def kernel(dst_nodes, root_ts, root_edge_feat, memory, memory_ts, mail, mail_ts,
           W_ih, b_ih, W_hh, b_hh, time_w, time_b, tl_W, tl_b, ln_g, ln_b,
           ep_src_W, ep_src_b, ep_dst_W, ep_dst_b, ep_out_W, ep_out_b):
    dstn = dst_nodes.astype(_i32)
    dstB = dstn[:B2].reshape(B2, 1)
    dstC = dstn[:B2].reshape(B2 // JBLK, 1, JBLK)

    gmem, gmail, gmem_ts, gmail_ts = _k1(
        memory, mail, memory_ts, mail_ts, dstn)

    weights = (
        W_ih.T, (b_ih + b_hh).reshape(1, DE), W_hh.T,
        time_w.reshape(1, DT), time_b.reshape(1, DT),
        tl_W[:, 0].reshape(1, DE), tl_b.reshape(1, DE),
        ln_g.reshape(1, DE), ln_b.reshape(1, DE),
        ep_src_W.T, ep_src_b.reshape(1, DE),
        ep_dst_W.T, ep_dst_b.reshape(1, DE),
        ep_out_W.T, ep_out_b.reshape(1, 1),
    )
    (memb, mailb, memtsb, mailtsb, norm8k, mailrow, loser2,
     pos_scores, neg_scores) = _k02(
        memory, mail, memory_ts, mail_ts, gmem, gmail, gmem_ts, gmail_ts,
        root_ts, root_edge_feat, dstB, dstC, weights)

    premask = jnp.where(loser2.reshape(B2) == 0, dstn[:B2], -1).astype(_i32)

    new_memory, new_mail, new_memory_ts, new_mail_ts = _k3(
        dstn[:B2], premask, norm8k, gmail_ts[:B2], mailrow, root_ts,
        memb, mailb, memtsb, mailtsb)

    return (pos_scores, neg_scores, new_memory, new_memory_ts,
            new_mail, new_mail_ts)
